# Initial kernel scaffold; baseline (speedup 1.0000x reference)
import jax
import jax.numpy as jnp
from jax.experimental import pallas as pl


def kernel(x, edge_index, batch, W1, b1, W2, b2, fc_W, fc_b):
    raise NotImplementedError("write your pallas kernel here")


# speedup vs baseline: 7.4005x; 7.4005x over previous
import functools

import jax
import jax.numpy as jnp
from jax import lax
from jax.experimental import pallas as pl
from jax.experimental.pallas import tpu as pltpu
from jax.experimental.pallas import tpu_sc as plsc

f32 = jnp.float32
i32 = jnp.int32

N = 10000
NP = 10240
E = 160000
EP = 163840
ER = EP // 128
IN_DIM = 256
HID = 512
NG = 128
RB = 1024
NRB = NP // RB

_mesh = plsc.VectorSubcoreMesh(core_axis_name="c", subcore_axis_name="s")


@functools.partial(
    pl.kernel,
    out_type=jax.ShapeDtypeStruct((2, NP, 128), f32),
    mesh=_mesh,
    scratch_types=[
        pltpu.VMEM((40, 128), i32),
        pltpu.VMEM((128, 128), f32),
        pltpu.VMEM_SHARED((NP, 128), f32),
        pltpu.SemaphoreType.DMA,
    ],
)
def _deg_kernel(dst_hbm, ones_hbm, zeros_hbm, out_hbm, dst_v, ones_v, shared, sem):
    c = lax.axis_index("c")
    s = lax.axis_index("s")
    wid = c * 16 + s
    pltpu.sync_copy(dst_hbm.at[pl.ds(wid * 40, 40)], dst_v)
    pltpu.sync_copy(ones_hbm, ones_v)
    pltpu.sync_copy(zeros_hbm, shared.at[pl.ds(s * 640, 640)])
    plsc.subcore_barrier()

    @pl.loop(0, 10)
    def _(m):
        ds_ = [pltpu.async_copy(ones_v, shared.at[dst_v.at[m * 4 + b]],
                                sem, add=True)
               for b in range(4)]
        for d in ds_:
            d.wait()

    plsc.subcore_barrier()

    @pl.when(s == 0)
    def _():
        pltpu.sync_copy(shared, out_hbm.at[c])


def _make_agg(nchunk):
    cpc = nchunk // 2
    MEGA = 2
    EROWS = 16
    nsteps = ER // 16
    nsuper = nsteps // EROWS

    @functools.partial(
        pl.kernel,
        out_type=jax.ShapeDtypeStruct((nchunk, NP, 128), f32),
        mesh=_mesh,
        scratch_types=[
            pltpu.VMEM((EROWS, 128), i32),
            pltpu.VMEM((EROWS, 128), i32),
            pltpu.VMEM((MEGA, 128, 128), f32),
            pltpu.VMEM_SHARED((NP, 128), f32),
            pltpu.SemaphoreType.DMA,
            pltpu.SemaphoreType.DMA,
        ],
    )
    def agg(g_hbm, src_hbm, dst_hbm, out_hbm, src_v, dst_v, bufs, acc, sem_g, sem_s):
        c = lax.axis_index("c")
        s = lax.axis_index("s")
        rows0 = s * (NP // 16)
        for cc in range(cpc):
            k = c * cpc + cc
            pltpu.sync_copy(g_hbm.at[k].at[pl.ds(rows0, NP // 16)],
                            acc.at[pl.ds(rows0, NP // 16)])
            plsc.subcore_barrier()

            for o in range(nsuper):
                pltpu.sync_copy(src_hbm.at[pl.ds(s * nsteps + o * EROWS, EROWS)],
                                src_v)
                pltpu.sync_copy(dst_hbm.at[pl.ds(s * nsteps + o * EROWS, EROWS)],
                                dst_v)

                @pl.loop(0, EROWS // MEGA)
                def _(m):
                    base = m * MEGA
                    gd = [pltpu.async_copy(g_hbm.at[k].at[src_v.at[base + b]],
                                           bufs.at[b], sem_g)
                          for b in range(MEGA)]
                    for d in gd:
                        d.wait()
                    sd = [pltpu.async_copy(bufs.at[b],
                                           acc.at[dst_v.at[base + b]],
                                           sem_s, add=True)
                          for b in range(MEGA)]
                    for d in sd:
                        d.wait()

            plsc.subcore_barrier()
            pltpu.sync_copy(acc.at[pl.ds(rows0, NP // 16)],
                            out_hbm.at[k].at[pl.ds(rows0, NP // 16)])

    return agg


_agg2 = _make_agg(2)
_agg4 = _make_agg(4)


def _prep_body(x_ref, deg_ref, g_ref, db_ref):
    r = pl.program_id(0)
    dsum = deg_ref[0, :, :1] + deg_ref[1, :, :1]
    rows = r * RB + lax.broadcasted_iota(i32, (RB, 1), 0)
    d = jnp.where(rows < N, lax.rsqrt(dsum + 1.0), 0.0)
    for k in range(IN_DIM // 128):
        g_ref[k] = x_ref[:, k * 128:(k + 1) * 128] * d
    db_ref[...] = jnp.broadcast_to(d, (RB, 128))


def _prep_call(xp, deg):
    return pl.pallas_call(
        _prep_body,
        grid=(NRB,),
        in_specs=[
            pl.BlockSpec((RB, IN_DIM), lambda r: (r, 0)),
            pl.BlockSpec((2, RB, 128), lambda r: (0, r, 0)),
        ],
        out_specs=[
            pl.BlockSpec((IN_DIM // 128, RB, 128), lambda r: (0, r, 0)),
            pl.BlockSpec((RB, 128), lambda r: (r, 0)),
        ],
        out_shape=[
            jax.ShapeDtypeStruct((IN_DIM // 128, NP, 128), f32),
            jax.ShapeDtypeStruct((NP, 128), f32),
        ],
    )(xp, deg)


def _a1_body(s_ref, db_ref, w_ref, b_ref, out_ref):
    d = db_ref[...]
    t = jnp.zeros((RB, 128), f32)
    for k in range(2):
        t += jnp.dot(s_ref[k] * d, w_ref[k], preferred_element_type=f32)
    h = jnp.maximum(t + b_ref[...], 0.0)
    out_ref[0] = h * d


def _a1_call(s1, db, w1r, b1r):
    return pl.pallas_call(
        _a1_body,
        grid=(NRB, HID // 128),
        in_specs=[
            pl.BlockSpec((2, RB, 128), lambda r, n: (0, r, 0)),
            pl.BlockSpec((RB, 128), lambda r, n: (r, 0)),
            pl.BlockSpec((2, 128, 128), lambda r, n: (0, 0, n)),
            pl.BlockSpec((1, 128), lambda r, n: (0, n)),
        ],
        out_specs=pl.BlockSpec((1, RB, 128), lambda r, n: (n, r, 0)),
        out_shape=jax.ShapeDtypeStruct((HID // 128, NP, 128), f32),
    )(s1, db, w1r, b1r)


def _a2_body(s_ref, db_ref, w_ref, b_ref, fcw_ref, z_ref):
    d = db_ref[...]
    sd = [s_ref[k] * d for k in range(4)]
    z = jnp.zeros((RB, 128), f32)
    for n in range(4):
        t = jnp.zeros((RB, 128), f32)
        for k in range(4):
            t += jnp.dot(sd[k], w_ref[k, :, n * 128:(n + 1) * 128],
                         preferred_element_type=f32)
        h = jnp.maximum(t + b_ref[:, n * 128:(n + 1) * 128], 0.0)
        z += jnp.dot(h, fcw_ref[n], preferred_element_type=f32)
    col = lax.broadcasted_iota(i32, (RB, 128), 1)
    z_ref[...] = jnp.where(col == 127, 1.0, z)


def _a2_call(s2, db, w2r, b2r, fcw):
    return pl.pallas_call(
        _a2_body,
        grid=(NRB,),
        in_specs=[
            pl.BlockSpec((4, RB, 128), lambda r: (0, r, 0)),
            pl.BlockSpec((RB, 128), lambda r: (r, 0)),
            pl.BlockSpec((4, 128, HID), lambda r: (0, 0, 0)),
            pl.BlockSpec((1, HID), lambda r: (0, 0)),
            pl.BlockSpec((4, 128, 128), lambda r: (0, 0, 0)),
        ],
        out_specs=pl.BlockSpec((RB, 128), lambda r: (r, 0)),
        out_shape=jax.ShapeDtypeStruct((NP, 128), f32),
    )(s2, db, w2r, b2r, fcw)


def _final_body(z_ref, bat_ref, fcb_ref, out_ref, acc_ref):
    r = pl.program_id(0)

    @pl.when(r == 0)
    def _():
        acc_ref[...] = jnp.zeros((NG, 128), f32)

    gids = lax.broadcasted_iota(i32, (NG, 1), 0).astype(f32)
    oh = jnp.where(gids == bat_ref[...], 1.0, 0.0)
    acc_ref[...] += jnp.dot(oh, z_ref[...], preferred_element_type=f32)

    @pl.when(r == NRB - 1)
    def _():
        a = acc_ref[...]
        cnt = jnp.maximum(a[:, 127:128], 1.0)
        col = lax.broadcasted_iota(i32, (NG, 128), 1)
        logits = jnp.where(col < 10, a / cnt + fcb_ref[...], -1e30)
        m = jnp.max(logits, axis=1, keepdims=True)
        ssum = jnp.sum(jnp.exp(logits - m), axis=1, keepdims=True)
        out_ref[...] = logits - m - jnp.log(ssum)


def _final_call(z, batf, fcbp):
    return pl.pallas_call(
        _final_body,
        grid=(NRB,),
        in_specs=[
            pl.BlockSpec((RB, 128), lambda r: (r, 0)),
            pl.BlockSpec((1, RB), lambda r: (0, r)),
            pl.BlockSpec((1, 128), lambda r: (0, 0)),
        ],
        out_specs=pl.BlockSpec((NG, 128), lambda r: (0, 0)),
        out_shape=jax.ShapeDtypeStruct((NG, 128), f32),
        scratch_shapes=[pltpu.VMEM((NG, 128), f32)],
    )(z, batf, fcbp)


def kernel(x, edge_index, batch, W1, b1, W2, b2, fc_W, fc_b):
    xp = jnp.pad(x.astype(f32), ((0, NP - N), (0, 0)))
    src = edge_index[0].astype(i32)
    dst = edge_index[1].astype(i32)
    srcp = jnp.pad(src, (0, EP - E)).reshape(ER, 128)
    dstp = jnp.pad(dst, (0, EP - E), constant_values=NP - 1).reshape(ER, 128)
    batf = jnp.pad(batch.astype(f32), (0, NP - N),
                   constant_values=-1.0).reshape(1, NP)

    deg = _deg_kernel(dstp, jnp.ones((128, 128), f32),
                      jnp.zeros((640, 128), f32))

    g0, db = _prep_call(xp, deg)
    s1 = _agg2(g0, srcp, dstp)
    g1 = _a1_call(s1, db, W1.astype(f32).reshape(2, 128, HID),
                  b1.astype(f32).reshape(1, HID))
    s2 = _agg4(g1, srcp, dstp)
    fcw = jnp.pad(fc_W.astype(f32), ((0, 0), (0, 128 - 10))).reshape(4, 128, 128)
    z = _a2_call(s2, db, W2.astype(f32).reshape(4, 128, HID),
                 b2.astype(f32).reshape(1, HID), fcw)
    out = _final_call(z, batf, jnp.pad(fc_b.astype(f32), (0, 128 - 10)).reshape(1, 128))
    return out[:, :10]
